# baseline (speedup 1.0000x reference)
import functools

import jax
import jax.numpy as jnp
import numpy as np
from jax import lax
from jax.experimental import pallas as pl
from jax.experimental.pallas import tpu as pltpu
from jax.experimental.pallas import tpu_sc as plsc

N_KNOTS = 1024
X_MIN = -0.5
X_MAX = 1.5
_DX = (X_MAX - X_MIN) / (N_KNOTS - 1)
_INV_DX = float(1.0 / (_DX + 1e-12))
_U_OFF = float(-X_MIN * _INV_DX)
_U_MAX = float(np.nextafter(np.float32(N_KNOTS - 1), np.float32(0.0)))

_ROWS = 8
_COLS = 128


def _knot_body(raw_ref, out_ref):
    raw = raw_ref[...]
    inc = jnp.maximum(raw, 0.0) + jnp.log(1.0 + jnp.exp(-jnp.abs(raw))) + 1e-06
    r = lax.broadcasted_iota(jnp.int32, (_COLS, _COLS), 0)
    c = lax.broadcasted_iota(jnp.int32, (_COLS, _COLS), 1)
    tri = (r <= c).astype(jnp.float32)
    row_cum = jnp.dot(inc, tri, preferred_element_type=jnp.float32)
    rr = lax.broadcasted_iota(jnp.int32, (_ROWS, _ROWS), 0)
    cc = lax.broadcasted_iota(jnp.int32, (_ROWS, _ROWS), 1)
    stri = (cc < rr).astype(jnp.float32)
    row_tot = jnp.broadcast_to(row_cum[:, _COLS - 1 :], (_ROWS, _COLS))
    row_off = jnp.dot(stri, row_tot, preferred_element_type=jnp.float32)
    cum = row_cum + row_off
    c0 = cum[0:1, 0:1]
    denom = cum[_ROWS - 1 :, _COLS - 1 :] - c0
    vals = (cum - c0) / (denom + 1e-12)
    col127 = jnp.concatenate([vals[1:, 0:1], vals[_ROWS - 1 :, 0:1]], axis=0)
    nxt = jnp.concatenate([vals[:, 1:], col127], axis=1)
    dy = nxt - vals
    bl = lax.bitcast_convert_type(vals, jnp.int32)
    bd = lax.bitcast_convert_type(dy, jnp.int32)
    lo = (bl + 0x7FFF + ((bl >> 16) & 1)) >> 16
    hi = (bd + 0x7FFF + ((bd >> 16) & 1)) & jnp.int32(-65536)
    out_ref[...] = lo | hi


def _knot_table(raw_increments):
    raw2d = raw_increments.reshape(_ROWS, _COLS)
    vals = pl.pallas_call(
        _knot_body,
        out_shape=jax.ShapeDtypeStruct((_ROWS, _COLS), jnp.int32),
    )(raw2d)
    return vals.reshape(N_KNOTS)


def _make_interp(n_rows, n_cols):
    info = plsc.get_sparse_core_info()
    nc, ns, lanes = info.num_cores, info.num_subcores, info.num_lanes
    nw = nc * ns
    rows_w = n_rows // nw
    assert rows_w * nw == n_rows

    rows_c = 64
    assert rows_w % rows_c == 0
    n_chunks = rows_w // rows_c
    offs = list(range(0, n_cols - lanes + 1, lanes))
    if offs[-1] + lanes < n_cols:
        offs.append(n_cols - lanes)
    assert all(o // 128 == (o + lanes - 1) // 128 for o in offs)
    mesh = plsc.VectorSubcoreMesh(core_axis_name="c", subcore_axis_name="s")

    @functools.partial(
        pl.kernel,
        mesh=mesh,
        out_type=jax.ShapeDtypeStruct((n_rows, n_cols), jnp.float32),
        compiler_params=pltpu.CompilerParams(
            needs_layout_passes=False, use_tc_tiling_on_sc=True
        ),
        scratch_types=[
            pltpu.VMEM((N_KNOTS,), jnp.int32),
            pltpu.VMEM((2, rows_c, n_cols), jnp.float32),
            pltpu.VMEM((2, rows_c, n_cols), jnp.float32),
            pltpu.SemaphoreType.DMA,
            pltpu.SemaphoreType.DMA,
            pltpu.SemaphoreType.DMA,
            pltpu.SemaphoreType.DMA,
        ],
    )
    def interp(x_hbm, ky_hbm, out_hbm, ky_v, xbuf, ybuf, is0, is1, os0, os1):
        wid = lax.axis_index("s") * nc + lax.axis_index("c")
        base = wid * rows_w
        in_sems = (is0, is1)
        out_sems = (os0, os1)
        ky_copy = pltpu.make_async_copy(ky_hbm, ky_v, os0)
        ky_copy.start()

        def in_copy(k, slot):
            return pltpu.make_async_copy(
                x_hbm.at[pl.ds(base + k * rows_c, rows_c), :],
                xbuf.at[slot],
                in_sems[slot],
            )

        def out_copy(k, slot):
            return pltpu.make_async_copy(
                ybuf.at[slot],
                out_hbm.at[pl.ds(base + k * rows_c, rows_c), :],
                out_sems[slot],
            )

        def compute(slot):
            @plsc.parallel_loop(0, rows_c, step=1, unroll=3)
            def body(r):
                for off in offs:
                    xv = xbuf[slot, r, pl.ds(off, lanes)]
                    u = xv * _INV_DX + _U_OFF
                    u = jnp.clip(u, 0.0, _U_MAX)
                    idx = u.astype(jnp.int32)
                    t = u - idx.astype(jnp.float32)
                    w = plsc.load_gather(ky_v, [idx])
                    y_l = plsc.bitcast(w << 16, jnp.float32)
                    dy = plsc.bitcast(w & jnp.int32(-65536), jnp.float32)
                    ybuf[slot, r, pl.ds(off, lanes)] = y_l + t * dy

        in_copy(0, 0).start()
        in_copy(1, 1).start()
        ky_copy.wait()

        def pair_body(p, _):
            for b in range(2):
                k = p * 2 + b
                in_copy(k, b).wait()

                @pl.when(k >= 2)
                def _():
                    out_copy(k - 2, b).wait()

                compute(b)
                out_copy(k, b).start()

                @pl.when(k + 2 < n_chunks)
                def _():
                    in_copy(k + 2, b).start()

            return 0

        assert n_chunks % 2 == 0 and n_chunks >= 2
        lax.fori_loop(0, n_chunks // 2, pair_body, 0)
        out_copy(n_chunks - 2, 0).wait()
        out_copy(n_chunks - 1, 1).wait()

    return interp


def kernel(x, raw_increments):
    ky = _knot_table(raw_increments)
    interp = _make_interp(x.shape[0], x.shape[1])
    return interp(x, ky)


# speedup vs baseline: 1.0028x; 1.0028x over previous
import functools

import jax
import jax.numpy as jnp
import numpy as np
from jax import lax
from jax.experimental import pallas as pl
from jax.experimental.pallas import tpu as pltpu
from jax.experimental.pallas import tpu_sc as plsc

N_KNOTS = 1024
X_MIN = -0.5
X_MAX = 1.5
_DX = (X_MAX - X_MIN) / (N_KNOTS - 1)
_INV_DX = float(1.0 / (_DX + 1e-12))
_U_OFF = float(-X_MIN * _INV_DX)
_U_MAX = float(np.nextafter(np.float32(N_KNOTS - 1), np.float32(0.0)))

_ROWS = 8
_COLS = 128


def _knot_body(raw_ref, out_ref):
    raw = raw_ref[...]
    inc = jnp.maximum(raw, 0.0) + jnp.log(1.0 + jnp.exp(-jnp.abs(raw))) + 1e-06
    r = lax.broadcasted_iota(jnp.int32, (_COLS, _COLS), 0)
    c = lax.broadcasted_iota(jnp.int32, (_COLS, _COLS), 1)
    tri = (r <= c).astype(jnp.float32)
    row_cum = jnp.dot(inc, tri, preferred_element_type=jnp.float32)
    rr = lax.broadcasted_iota(jnp.int32, (_ROWS, _ROWS), 0)
    cc = lax.broadcasted_iota(jnp.int32, (_ROWS, _ROWS), 1)
    stri = (cc < rr).astype(jnp.float32)
    row_tot = jnp.broadcast_to(row_cum[:, _COLS - 1 :], (_ROWS, _COLS))
    row_off = jnp.dot(stri, row_tot, preferred_element_type=jnp.float32)
    cum = row_cum + row_off
    c0 = cum[0:1, 0:1]
    denom = cum[_ROWS - 1 :, _COLS - 1 :] - c0
    vals = (cum - c0) / (denom + 1e-12)
    col127 = jnp.concatenate([vals[1:, 0:1], vals[_ROWS - 1 :, 0:1]], axis=0)
    nxt = jnp.concatenate([vals[:, 1:], col127], axis=1)
    dy = nxt - vals
    bl = lax.bitcast_convert_type(vals, jnp.int32)
    bd = lax.bitcast_convert_type(dy, jnp.int32)
    lo = (bl + 0x7FFF + ((bl >> 16) & 1)) >> 16
    hi = (bd + 0x7FFF + ((bd >> 16) & 1)) & jnp.int32(-65536)
    out_ref[...] = lo | hi


def _knot_table(raw_increments):
    raw2d = raw_increments.reshape(_ROWS, _COLS)
    vals = pl.pallas_call(
        _knot_body,
        out_shape=jax.ShapeDtypeStruct((_ROWS, _COLS), jnp.int32),
    )(raw2d)
    return vals.reshape(N_KNOTS)


def _make_interp(n_rows, n_cols):
    info = plsc.get_sparse_core_info()
    nc, ns, lanes = info.num_cores, info.num_subcores, info.num_lanes
    nw = nc * ns
    rows_w = n_rows // nw
    assert rows_w * nw == n_rows

    rows_c = 64
    assert rows_w % rows_c == 0
    n_chunks = rows_w // rows_c
    offs = list(range(0, n_cols - lanes + 1, lanes))
    if offs[-1] + lanes < n_cols:
        offs.append(n_cols - lanes)
    assert all(o // 128 == (o + lanes - 1) // 128 for o in offs)
    mesh = plsc.VectorSubcoreMesh(core_axis_name="c", subcore_axis_name="s")

    @functools.partial(
        pl.kernel,
        mesh=mesh,
        out_type=jax.ShapeDtypeStruct((n_rows, n_cols), jnp.float32),
        compiler_params=pltpu.CompilerParams(
            needs_layout_passes=False, use_tc_tiling_on_sc=True
        ),
        scratch_types=[
            pltpu.VMEM((N_KNOTS,), jnp.int32),
            pltpu.VMEM((2, rows_c, n_cols), jnp.float32),
            pltpu.VMEM((2, rows_c, n_cols), jnp.float32),
            pltpu.SemaphoreType.DMA,
            pltpu.SemaphoreType.DMA,
            pltpu.SemaphoreType.DMA,
            pltpu.SemaphoreType.DMA,
        ],
    )
    def interp(x_hbm, ky_hbm, out_hbm, ky_v, xbuf, ybuf, is0, is1, os0, os1):
        wid = lax.axis_index("s") * nc + lax.axis_index("c")
        base = wid * rows_w
        in_sems = (is0, is1)
        out_sems = (os0, os1)
        ky_copy = pltpu.make_async_copy(ky_hbm, ky_v, os0)
        ky_copy.start()

        def in_copy(k, slot):
            return pltpu.make_async_copy(
                x_hbm.at[pl.ds(base + k * rows_c, rows_c), :],
                xbuf.at[slot],
                in_sems[slot],
            )

        def out_copy(k, slot):
            return pltpu.make_async_copy(
                ybuf.at[slot],
                out_hbm.at[pl.ds(base + k * rows_c, rows_c), :],
                out_sems[slot],
            )

        def compute(slot):
            @plsc.parallel_loop(0, rows_c, step=1, unroll=2)
            def body(r):
                for off in offs:
                    xv = xbuf[slot, r, pl.ds(off, lanes)]
                    u = xv * _INV_DX + _U_OFF
                    u = jnp.clip(u, 0.0, _U_MAX)
                    idx = u.astype(jnp.int32)
                    t = u - idx.astype(jnp.float32)
                    w = plsc.load_gather(ky_v, [idx])
                    y_l = plsc.bitcast(w << 16, jnp.float32)
                    dy = plsc.bitcast(w & jnp.int32(-65536), jnp.float32)
                    ybuf[slot, r, pl.ds(off, lanes)] = y_l + t * dy

        in_copy(0, 0).start()
        in_copy(1, 1).start()
        ky_copy.wait()

        def pair_body(p, _):
            for b in range(2):
                k = p * 2 + b
                in_copy(k, b).wait()

                @pl.when(k >= 2)
                def _():
                    out_copy(k - 2, b).wait()

                compute(b)
                out_copy(k, b).start()

                @pl.when(k + 2 < n_chunks)
                def _():
                    in_copy(k + 2, b).start()

            return 0

        assert n_chunks % 2 == 0 and n_chunks >= 2
        lax.fori_loop(0, n_chunks // 2, pair_body, 0)
        out_copy(n_chunks - 2, 0).wait()
        out_copy(n_chunks - 1, 1).wait()

    return interp


def kernel(x, raw_increments):
    ky = _knot_table(raw_increments)
    interp = _make_interp(x.shape[0], x.shape[1])
    return interp(x, ky)
